# baseline (speedup 1.0000x reference)
import jax
import jax.numpy as jnp
from jax import lax
from jax.experimental import pallas as pl
from jax.experimental.pallas import tpu as pltpu
from jax.experimental.pallas import tpu_sc as plsc

N = 10000
NP = 10240
NT = 640
FP = 8
FC = 4
E = 320000
EP = 327680
ET = 20480
CH = 2048
NCH = ET // CH
ALPHA = 0.1
ITERS = 10
DF = 128
DH = 64
NS = 16


def _mlp_body(x_ref, w1_ref, b1_ref, w2_ref, b2_ref, o_ref):
  h = jnp.dot(x_ref[...], w1_ref[...], preferred_element_type=jnp.float32)
  h = jnp.maximum(h + b1_ref[...], 0.0)
  o = jnp.dot(h, w2_ref[...], preferred_element_type=jnp.float32)
  o_ref[...] = o + b2_ref[...]


def _mlp(x_p, W1, b1, W2p, b2p):
  BM = 1024
  return pl.pallas_call(
      _mlp_body,
      grid=(NP // BM,),
      in_specs=[
          pl.BlockSpec((BM, DF), lambda i: (i, 0)),
          pl.BlockSpec((DF, DH), lambda i: (0, 0)),
          pl.BlockSpec((1, DH), lambda i: (0, 0)),
          pl.BlockSpec((DH, FP), lambda i: (0, 0)),
          pl.BlockSpec((1, FP), lambda i: (0, 0)),
      ],
      out_specs=pl.BlockSpec((BM, FP), lambda i: (i, 0)),
      out_shape=jax.ShapeDtypeStruct((NP, FP), jnp.float32),
  )(x_p, W1, b1.reshape(1, DH), W2p, b2p.reshape(1, FP))


def _rsqrt_newton(x):
  y = 1.0 / x
  for _ in range(28):
    y = y * (1.5 - 0.5 * x * y * y)
  return y


def _appnp_body(h0_hbm, rc_hbm, w_hbm, out_hbm, p_hbm,
                rcb0, rcb1, wb0, wb1, rowb,
                ul0, ul1, ul2, ul3, al0, al1, al2, al3,
                rbuf, ob0, ob1, ob2, ob3, hw0, hw1, hw2, hw3,
                hbuf, dis_v, dis2_v,
                ld0, ld1, psem, rsem,
                u_s0, u_s1, u_s2, u_s3, deg_s, dis_s):
  rcb = [rcb0, rcb1]
  wb = [wb0, wb1]
  u_loc = [ul0, ul1, ul2, ul3]
  a_loc = [al0, al1, al2, al3]
  h0buf = [ob0, ob1, ob2, ob3]
  hown = [hw0, hw1, hw2, hw3]
  u_s = [u_s0, u_s1, u_s2, u_s3]
  lsem = [ld0, ld1]
  cid = lax.axis_index("c")
  sid = lax.axis_index("s")
  base_n = sid * NT
  own = pl.ds(base_n, NT)

  zeros16 = jnp.zeros((16,), jnp.float32)

  def fire_load(c, b):
    return [pltpu.async_copy(rc_hbm.at[sid, c], rcb[b], lsem[b]),
            pltpu.async_copy(w_hbm.at[sid, c], wb[b], lsem[b])]

  def azero(v, _):
    for f in range(FC):
      a_loc[f][pl.ds(16 * v, 16)] = zeros16
    return 0
  lax.fori_loop(0, NP // 16, azero, 0)

  for f in range(FC):
    pltpu.sync_copy(h0_hbm.at[cid, f, own], h0buf[f])
  pltpu.sync_copy(al0.at[pl.ds(0, NT)], deg_s.at[own])
  plsc.subcore_barrier()

  pd = {0: fire_load(0, 0)}
  for c in range(NCH):
    if c + 1 < NCH:
      pd[c + 1] = fire_load(c + 1, (c + 1) % 2)
    for d in pd.pop(c):
      d.wait()
    b = c % 2

    def rowex(v, _):
      rowb[pl.ds(16 * v, 16)] = lax.shift_right_logical(
          rcb[b][pl.ds(16 * v, 16)], 14)
      return 0
    lax.fori_loop(0, CH // 16, rowex, 0)
    pltpu.sync_copy(wb[b], deg_s.at[rowb], add=True)
  plsc.subcore_barrier()

  pltpu.sync_copy(deg_s.at[own], hbuf)

  def dis_calc(v, _):
    dv = hbuf[pl.ds(16 * v, 16)] + 1.0
    d = _rsqrt_newton(dv)
    dis_v[pl.ds(16 * v, 16)] = d
    dis2_v[pl.ds(16 * v, 16)] = d * d
    return 0
  lax.fori_loop(0, NT // 16, dis_calc, 0)

  pltpu.sync_copy(dis_v, dis_s.at[own])

  for f in range(FC):
    def u_init(v, _):
      ds16 = pl.ds(16 * v, 16)
      h0v = h0buf[f][ds16]
      hown[f][ds16] = h0v
      hbuf[ds16] = dis_v[ds16] * h0v
      return 0
    lax.fori_loop(0, NT // 16, u_init, 0)
    pltpu.sync_copy(hbuf, u_s[f].at[own])
  plsc.subcore_barrier()

  for f in range(FC):
    pltpu.sync_copy(u_s[f], u_loc[f])

  def one_iter(_, carry):
    pd = {0: fire_load(0, 0)}
    for c in range(NCH):
      if c + 1 < NCH:
        pd[c + 1] = fire_load(c + 1, (c + 1) % 2)
      for d in pd.pop(c):
        d.wait()
      b = c % 2

      def edge(v0, _):
        for u in range(4):
          ds16 = pl.ds(64 * v0 + 16 * u, 16)
          rc = rcb[b][ds16]
          wv = wb[b][ds16]
          row = lax.shift_right_logical(rc, 14)
          col = lax.bitwise_and(rc, 16383)
          for f in range(FC):
            uv = plsc.load_gather(u_loc[f], [col])
            plsc.addupdate_scatter(a_loc[f], [row], uv * wv)
        return 0
      lax.fori_loop(0, CH // 64, edge, 0)

    wd = [pltpu.async_copy(a_loc[f], p_hbm.at[cid, f, sid], psem)
          for f in range(FC)]
    for d in wd:
      d.wait()
    plsc.subcore_barrier()

    for f in range(FC):
      rd = [pltpu.async_copy(p_hbm.at[cid, f, t, pl.ds(base_n, NT)],
                             rbuf.at[pl.ds(t * NT, NT)], rsem)
            for t in range(NS)]
      for d in rd:
        d.wait()

      def red(v, _):
        ds16 = pl.ds(16 * v, 16)
        a = rbuf[pl.ds(16 * v, 16)]
        for t in range(1, NS):
          a = a + rbuf[pl.ds(t * NT + 16 * v, 16)]
        hn = ((1.0 - ALPHA) * (dis_v[ds16] * a
                               + dis2_v[ds16] * hown[f][ds16])
              + ALPHA * h0buf[f][ds16])
        hown[f][ds16] = hn
        hbuf[ds16] = dis_v[ds16] * hn
        return 0
      lax.fori_loop(0, NT // 16, red, 0)
      pltpu.sync_copy(hbuf, u_s[f].at[own])

    def azero2(v, _):
      for f in range(FC):
        a_loc[f][pl.ds(16 * v, 16)] = zeros16
      return 0
    lax.fori_loop(0, NP // 16, azero2, 0)
    plsc.subcore_barrier()

    for f in range(FC):
      pltpu.sync_copy(u_s[f], u_loc[f])
    return carry

  lax.fori_loop(0, ITERS, one_iter, 0)

  for f in range(FC):
    pltpu.sync_copy(hown[f], out_hbm.at[cid, f, own])


def _appnp(h0_t, rc_p, w_p):
  mesh = plsc.VectorSubcoreMesh(core_axis_name="c", subcore_axis_name="s",
                                num_cores=2, num_subcores=16)
  f = pl.kernel(
      _appnp_body,
      out_type=[jax.ShapeDtypeStruct((2, FC, NP), jnp.float32),
                jax.ShapeDtypeStruct((2, FC, NS, NP), jnp.float32)],
      mesh=mesh,
      compiler_params=pltpu.CompilerParams(needs_layout_passes=False),
      scratch_types=[
          pltpu.VMEM((CH,), jnp.int32),
          pltpu.VMEM((CH,), jnp.int32),
          pltpu.VMEM((CH,), jnp.float32),
          pltpu.VMEM((CH,), jnp.float32),
          pltpu.VMEM((CH,), jnp.int32),
          pltpu.VMEM((NP,), jnp.float32),
          pltpu.VMEM((NP,), jnp.float32),
          pltpu.VMEM((NP,), jnp.float32),
          pltpu.VMEM((NP,), jnp.float32),
          pltpu.VMEM((NP,), jnp.float32),
          pltpu.VMEM((NP,), jnp.float32),
          pltpu.VMEM((NP,), jnp.float32),
          pltpu.VMEM((NP,), jnp.float32),
          pltpu.VMEM((NS * NT,), jnp.float32),
          pltpu.VMEM((NT,), jnp.float32),
          pltpu.VMEM((NT,), jnp.float32),
          pltpu.VMEM((NT,), jnp.float32),
          pltpu.VMEM((NT,), jnp.float32),
          pltpu.VMEM((NT,), jnp.float32),
          pltpu.VMEM((NT,), jnp.float32),
          pltpu.VMEM((NT,), jnp.float32),
          pltpu.VMEM((NT,), jnp.float32),
          pltpu.VMEM((NT,), jnp.float32),
          pltpu.VMEM((NT,), jnp.float32),
          pltpu.VMEM((NT,), jnp.float32),
          pltpu.SemaphoreType.DMA,
          pltpu.SemaphoreType.DMA,
          pltpu.SemaphoreType.DMA,
          pltpu.SemaphoreType.DMA,
          pltpu.VMEM_SHARED((NP,), jnp.float32),
          pltpu.VMEM_SHARED((NP,), jnp.float32),
          pltpu.VMEM_SHARED((NP,), jnp.float32),
          pltpu.VMEM_SHARED((NP,), jnp.float32),
          pltpu.VMEM_SHARED((NP,), jnp.float32),
          pltpu.VMEM_SHARED((NP,), jnp.float32),
      ],
  )
  return f(h0_t, rc_p, w_p)


@jax.jit
def kernel(x, edge_index, edge_weight, W1, b1, W2, b2):
  x_p = jnp.pad(x, ((0, NP - N), (0, 0)))
  W2p = jnp.pad(W2, ((0, 0), (0, FP - W2.shape[1])))
  b2p = jnp.pad(b2, (0, FP - b2.shape[0]))

  h0 = _mlp(x_p, W1, b1, W2p, b2p)
  h0_t = h0.T.reshape(2, FC, NP)

  npad = EP - E
  pad_idx = (jnp.arange(npad, dtype=jnp.int32) * 131) % N
  row_p = jnp.concatenate([edge_index[0], pad_idx])
  col_p = jnp.concatenate([edge_index[1], pad_idx])
  rc_p = (row_p * 16384 + col_p).reshape(16, NCH, CH)
  w_p = jnp.concatenate(
      [edge_weight, jnp.zeros((npad,), jnp.float32)]).reshape(16, NCH, CH)

  out, _partials = _appnp(h0_t, rc_p, w_p)
  out = out.reshape(FP, NP).T
  return out[:N, :6]


# speedup vs baseline: 1.6891x; 1.6891x over previous
import jax
import jax.numpy as jnp
from jax import lax
from jax.experimental import pallas as pl
from jax.experimental.pallas import tpu as pltpu
from jax.experimental.pallas import tpu_sc as plsc

N = 10000
NP = 10240
NT = 640
FP = 8
FC = 4
E = 320000
EP = 327680
ET = 20480
CH = 2048
NCH = ET // CH
ALPHA = 0.1
ITERS = 10
DF = 128
DH = 64
NS = 16


def _mlp_body(x_ref, w1_ref, b1_ref, w2_ref, b2_ref, o_ref):
  h = jnp.dot(x_ref[...], w1_ref[...], preferred_element_type=jnp.float32)
  h = jnp.maximum(h + b1_ref[...], 0.0)
  o = jnp.dot(h, w2_ref[...], preferred_element_type=jnp.float32)
  o_ref[...] = o + b2_ref[...]


def _mlp(x_p, W1, b1, W2p, b2p):
  BM = 1024
  return pl.pallas_call(
      _mlp_body,
      grid=(NP // BM,),
      in_specs=[
          pl.BlockSpec((BM, DF), lambda i: (i, 0)),
          pl.BlockSpec((DF, DH), lambda i: (0, 0)),
          pl.BlockSpec((1, DH), lambda i: (0, 0)),
          pl.BlockSpec((DH, FP), lambda i: (0, 0)),
          pl.BlockSpec((1, FP), lambda i: (0, 0)),
      ],
      out_specs=pl.BlockSpec((BM, FP), lambda i: (i, 0)),
      out_shape=jax.ShapeDtypeStruct((NP, FP), jnp.float32),
  )(x_p, W1, b1.reshape(1, DH), W2p, b2p.reshape(1, FP))


def _rsqrt_newton(x):
  y = 1.0 / x
  for _ in range(28):
    y = y * (1.5 - 0.5 * x * y * y)
  return y


def _appnp_body(h0_hbm, rc_hbm, w_hbm, out_hbm, p_hbm,
                rcb0, rcb1, wb0, wb1, rowb,
                ul0, ul1, ul2, ul3, al0, al1, al2, al3,
                rbuf, ob0, ob1, ob2, ob3, hw0, hw1, hw2, hw3,
                hbuf, dis_v, dis2_v,
                ld0, ld1, psem, rsem,
                u_s0, u_s1, u_s2, u_s3, deg_s, dis_s):
  rcb = [rcb0, rcb1]
  wb = [wb0, wb1]
  u_loc = [ul0, ul1, ul2, ul3]
  a_loc = [al0, al1, al2, al3]
  h0buf = [ob0, ob1, ob2, ob3]
  hown = [hw0, hw1, hw2, hw3]
  u_s = [u_s0, u_s1, u_s2, u_s3]
  lsem = [ld0, ld1]
  cid = lax.axis_index("c")
  sid = lax.axis_index("s")
  base_n = sid * NT
  own = pl.ds(base_n, NT)

  zeros16 = jnp.zeros((16,), jnp.float32)

  def fire_load(c, b):
    return [pltpu.async_copy(rc_hbm.at[sid, c], rcb[b], lsem[b]),
            pltpu.async_copy(w_hbm.at[sid, c], wb[b], lsem[b])]

  def azero(v, _):
    for f in range(FC):
      a_loc[f][pl.ds(16 * v, 16)] = zeros16
    return 0
  lax.fori_loop(0, NP // 16, azero, 0)

  for f in range(FC):
    pltpu.sync_copy(h0_hbm.at[cid, f, own], h0buf[f])
  pltpu.sync_copy(al0.at[pl.ds(0, NT)], deg_s.at[own])
  plsc.subcore_barrier()

  pd = {0: fire_load(0, 0)}
  for c in range(NCH):
    if c + 1 < NCH:
      pd[c + 1] = fire_load(c + 1, (c + 1) % 2)
    for d in pd.pop(c):
      d.wait()
    b = c % 2

    def rowex(v, _):
      rowb[pl.ds(16 * v, 16)] = lax.shift_right_logical(
          rcb[b][pl.ds(16 * v, 16)], 14)
      return 0
    lax.fori_loop(0, CH // 16, rowex, 0)
    pltpu.sync_copy(wb[b], deg_s.at[rowb], add=True)
  plsc.subcore_barrier()

  pltpu.sync_copy(deg_s.at[own], hbuf)

  def dis_calc(v, _):
    dv = hbuf[pl.ds(16 * v, 16)] + 1.0
    d = _rsqrt_newton(dv)
    dis_v[pl.ds(16 * v, 16)] = d
    dis2_v[pl.ds(16 * v, 16)] = d * d
    return 0
  lax.fori_loop(0, NT // 16, dis_calc, 0)

  pltpu.sync_copy(dis_v, dis_s.at[own])

  for f in range(FC):
    def u_init(v, _):
      ds16 = pl.ds(16 * v, 16)
      h0v = h0buf[f][ds16]
      hown[f][ds16] = h0v
      hbuf[ds16] = dis_v[ds16] * h0v
      return 0
    lax.fori_loop(0, NT // 16, u_init, 0)
    pltpu.sync_copy(hbuf, u_s[f].at[own])
  plsc.subcore_barrier()

  for f in range(FC):
    pltpu.sync_copy(u_s[f], u_loc[f])

  def one_iter(_, carry):
    pd = {0: fire_load(0, 0)}
    for c in range(NCH):
      if c + 1 < NCH:
        pd[c + 1] = fire_load(c + 1, (c + 1) % 2)
      for d in pd.pop(c):
        d.wait()
      b = c % 2

      @plsc.parallel_loop(0, CH // 64, unroll=2)
      def edge(v0):
        for u in range(4):
          ds16 = pl.ds(64 * v0 + 16 * u, 16)
          rc = rcb[b][ds16]
          wv = wb[b][ds16]
          row = lax.shift_right_logical(rc, 14)
          col = lax.bitwise_and(rc, 16383)
          for f in range(FC):
            uv = plsc.load_gather(u_loc[f], [col])
            plsc.addupdate_scatter(a_loc[f], [row], uv * wv)

    wd = [pltpu.async_copy(a_loc[f], p_hbm.at[cid, f, sid], psem)
          for f in range(FC)]
    for d in wd:
      d.wait()

    def azero2(v, _):
      for f in range(FC):
        a_loc[f][pl.ds(16 * v, 16)] = zeros16
      return 0
    lax.fori_loop(0, NP // 16, azero2, 0)
    plsc.subcore_barrier()

    for f in range(FC):
      rd = [pltpu.async_copy(p_hbm.at[cid, f, t, pl.ds(base_n, NT)],
                             rbuf.at[pl.ds(t * NT, NT)], rsem)
            for t in range(NS)]
      for d in rd:
        d.wait()

      def red(v, _):
        ds16 = pl.ds(16 * v, 16)
        a = rbuf[pl.ds(16 * v, 16)]
        for t in range(1, NS):
          a = a + rbuf[pl.ds(t * NT + 16 * v, 16)]
        hn = ((1.0 - ALPHA) * (dis_v[ds16] * a
                               + dis2_v[ds16] * hown[f][ds16])
              + ALPHA * h0buf[f][ds16])
        hown[f][ds16] = hn
        hbuf[ds16] = dis_v[ds16] * hn
        return 0
      lax.fori_loop(0, NT // 16, red, 0)
      pltpu.sync_copy(hbuf, u_s[f].at[own])

    plsc.subcore_barrier()

    for f in range(FC):
      pltpu.sync_copy(u_s[f], u_loc[f])
    return carry

  lax.fori_loop(0, ITERS, one_iter, 0)

  for f in range(FC):
    pltpu.sync_copy(hown[f], out_hbm.at[cid, f, own])


def _appnp(h0_t, rc_p, w_p):
  mesh = plsc.VectorSubcoreMesh(core_axis_name="c", subcore_axis_name="s",
                                num_cores=2, num_subcores=16)
  f = pl.kernel(
      _appnp_body,
      out_type=[jax.ShapeDtypeStruct((2, FC, NP), jnp.float32),
                jax.ShapeDtypeStruct((2, FC, NS, NP), jnp.float32)],
      mesh=mesh,
      compiler_params=pltpu.CompilerParams(needs_layout_passes=False),
      scratch_types=[
          pltpu.VMEM((CH,), jnp.int32),
          pltpu.VMEM((CH,), jnp.int32),
          pltpu.VMEM((CH,), jnp.float32),
          pltpu.VMEM((CH,), jnp.float32),
          pltpu.VMEM((CH,), jnp.int32),
          pltpu.VMEM((NP,), jnp.float32),
          pltpu.VMEM((NP,), jnp.float32),
          pltpu.VMEM((NP,), jnp.float32),
          pltpu.VMEM((NP,), jnp.float32),
          pltpu.VMEM((NP,), jnp.float32),
          pltpu.VMEM((NP,), jnp.float32),
          pltpu.VMEM((NP,), jnp.float32),
          pltpu.VMEM((NP,), jnp.float32),
          pltpu.VMEM((NS * NT,), jnp.float32),
          pltpu.VMEM((NT,), jnp.float32),
          pltpu.VMEM((NT,), jnp.float32),
          pltpu.VMEM((NT,), jnp.float32),
          pltpu.VMEM((NT,), jnp.float32),
          pltpu.VMEM((NT,), jnp.float32),
          pltpu.VMEM((NT,), jnp.float32),
          pltpu.VMEM((NT,), jnp.float32),
          pltpu.VMEM((NT,), jnp.float32),
          pltpu.VMEM((NT,), jnp.float32),
          pltpu.VMEM((NT,), jnp.float32),
          pltpu.VMEM((NT,), jnp.float32),
          pltpu.SemaphoreType.DMA,
          pltpu.SemaphoreType.DMA,
          pltpu.SemaphoreType.DMA,
          pltpu.SemaphoreType.DMA,
          pltpu.VMEM_SHARED((NP,), jnp.float32),
          pltpu.VMEM_SHARED((NP,), jnp.float32),
          pltpu.VMEM_SHARED((NP,), jnp.float32),
          pltpu.VMEM_SHARED((NP,), jnp.float32),
          pltpu.VMEM_SHARED((NP,), jnp.float32),
          pltpu.VMEM_SHARED((NP,), jnp.float32),
      ],
  )
  return f(h0_t, rc_p, w_p)


@jax.jit
def kernel(x, edge_index, edge_weight, W1, b1, W2, b2):
  x_p = jnp.pad(x, ((0, NP - N), (0, 0)))
  W2p = jnp.pad(W2, ((0, 0), (0, FP - W2.shape[1])))
  b2p = jnp.pad(b2, (0, FP - b2.shape[0]))

  h0 = _mlp(x_p, W1, b1, W2p, b2p)
  h0_t = h0.T.reshape(2, FC, NP)

  npad = EP - E
  pad_idx = (jnp.arange(npad, dtype=jnp.int32) * 131) % N
  row_p = jnp.concatenate([edge_index[0], pad_idx])
  col_p = jnp.concatenate([edge_index[1], pad_idx])
  rc_p = (row_p * 16384 + col_p).reshape(16, NCH, CH)
  w_p = jnp.concatenate(
      [edge_weight, jnp.zeros((npad,), jnp.float32)]).reshape(16, NCH, CH)

  out, _partials = _appnp(h0_t, rc_p, w_p)
  out = out.reshape(FP, NP).T
  return out[:N, :6]


# speedup vs baseline: 2.0672x; 1.2239x over previous
import jax
import jax.numpy as jnp
from jax import lax
from jax.experimental import pallas as pl
from jax.experimental.pallas import tpu as pltpu
from jax.experimental.pallas import tpu_sc as plsc

N = 10000
NP = 10240
NT = 640
FP = 8
FC = 4
E = 320000
EP = 327680
ET = 20480
CH = 2048
NCH = ET // CH
ALPHA = 0.1
ITERS = 10
DF = 128
DH = 64
NS = 16


def _mlp_body(x_ref, w1_ref, b1_ref, w2_ref, b2_ref, o_ref):
  h = jnp.dot(x_ref[...], w1_ref[...], preferred_element_type=jnp.float32)
  h = jnp.maximum(h + b1_ref[...], 0.0)
  o = jnp.dot(h, w2_ref[...], preferred_element_type=jnp.float32)
  o_ref[...] = o + b2_ref[...]


def _mlp(x_p, W1, b1, W2p, b2p):
  BM = 1024
  return pl.pallas_call(
      _mlp_body,
      grid=(NP // BM,),
      in_specs=[
          pl.BlockSpec((BM, DF), lambda i: (i, 0)),
          pl.BlockSpec((DF, DH), lambda i: (0, 0)),
          pl.BlockSpec((1, DH), lambda i: (0, 0)),
          pl.BlockSpec((DH, FP), lambda i: (0, 0)),
          pl.BlockSpec((1, FP), lambda i: (0, 0)),
      ],
      out_specs=pl.BlockSpec((BM, FP), lambda i: (i, 0)),
      out_shape=jax.ShapeDtypeStruct((NP, FP), jnp.float32),
  )(x_p, W1, b1.reshape(1, DH), W2p, b2p.reshape(1, FP))


def _rsqrt_newton(x):
  y = 1.0 / x
  for _ in range(28):
    y = y * (1.5 - 0.5 * x * y * y)
  return y


def _appnp_body(h0_hbm, rc_hbm, w_hbm, out_hbm, p_hbm,
                rcb0, rcb1, wb0, wb1, rowb,
                ul0, ul1, ul2, ul3, al0, al1, al2, al3,
                rbuf, rbuf2, ob0, ob1, ob2, ob3, hw0, hw1, hw2, hw3,
                hbuf, dis_v, dis2_v,
                ld0, ld1, psem, rsem,
                u_s0, u_s1, u_s2, u_s3, deg_s, dis_s):
  rcb = [rcb0, rcb1]
  wb = [wb0, wb1]
  u_loc = [ul0, ul1, ul2, ul3]
  a_loc = [al0, al1, al2, al3]
  h0buf = [ob0, ob1, ob2, ob3]
  hown = [hw0, hw1, hw2, hw3]
  u_s = [u_s0, u_s1, u_s2, u_s3]
  lsem = [ld0, ld1]
  cid = lax.axis_index("c")
  sid = lax.axis_index("s")
  base_n = sid * NT
  own = pl.ds(base_n, NT)

  zeros16 = jnp.zeros((16,), jnp.float32)

  def fire_load(c, b):
    return [pltpu.async_copy(rc_hbm.at[sid, c], rcb[b], lsem[b]),
            pltpu.async_copy(w_hbm.at[sid, c], wb[b], lsem[b])]

  @plsc.parallel_loop(0, NP // 16, unroll=4)
  def azero(v):
    for f in range(FC):
      a_loc[f][pl.ds(16 * v, 16)] = zeros16

  for f in range(FC):
    pltpu.sync_copy(h0_hbm.at[cid, f, own], h0buf[f])
  pltpu.sync_copy(al0.at[pl.ds(0, NT)], deg_s.at[own])
  plsc.subcore_barrier()

  pd = {0: fire_load(0, 0)}
  for c in range(NCH):
    if c + 1 < NCH:
      pd[c + 1] = fire_load(c + 1, (c + 1) % 2)
    for d in pd.pop(c):
      d.wait()
    b = c % 2

    @plsc.parallel_loop(0, CH // 16, unroll=4)
    def rowex(v):
      rowb[pl.ds(16 * v, 16)] = lax.shift_right_logical(
          rcb[b][pl.ds(16 * v, 16)], 14)
    pltpu.sync_copy(wb[b], deg_s.at[rowb], add=True)
  plsc.subcore_barrier()

  pltpu.sync_copy(deg_s.at[own], hbuf)

  def dis_calc(v, _):
    dv = hbuf[pl.ds(16 * v, 16)] + 1.0
    d = _rsqrt_newton(dv)
    dis_v[pl.ds(16 * v, 16)] = d
    dis2_v[pl.ds(16 * v, 16)] = d * d
    return 0
  lax.fori_loop(0, NT // 16, dis_calc, 0)

  pltpu.sync_copy(dis_v, dis_s.at[own])

  for f in range(FC):
    def u_init(v, _):
      ds16 = pl.ds(16 * v, 16)
      h0v = h0buf[f][ds16]
      hown[f][ds16] = h0v
      hbuf[ds16] = dis_v[ds16] * h0v
      return 0
    lax.fori_loop(0, NT // 16, u_init, 0)
    pltpu.sync_copy(hbuf, u_s[f].at[own])
  plsc.subcore_barrier()

  for f in range(FC):
    pltpu.sync_copy(u_s[f], u_loc[f])

  def one_iter(_, carry):
    pd = {0: fire_load(0, 0)}
    for c in range(NCH):
      if c + 1 < NCH:
        pd[c + 1] = fire_load(c + 1, (c + 1) % 2)
      for d in pd.pop(c):
        d.wait()
      b = c % 2

      @plsc.parallel_loop(0, CH // 64, unroll=2)
      def edge(v0):
        for u in range(4):
          ds16 = pl.ds(64 * v0 + 16 * u, 16)
          rc = rcb[b][ds16]
          wv = wb[b][ds16]
          row = lax.shift_right_logical(rc, 14)
          col = lax.bitwise_and(rc, 16383)
          for f in range(FC):
            uv = plsc.load_gather(u_loc[f], [col])
            plsc.addupdate_scatter(a_loc[f], [row], uv * wv)

    wd = [pltpu.async_copy(a_loc[f], p_hbm.at[cid, f, sid], psem)
          for f in range(FC)]
    for d in wd:
      d.wait()

    @plsc.parallel_loop(0, NP // 16, unroll=4)
    def azero2(v):
      for f in range(FC):
        a_loc[f][pl.ds(16 * v, 16)] = zeros16
    plsc.subcore_barrier()

    rb = [rbuf, rbuf2]

    def fire_red(f):
      return [pltpu.async_copy(p_hbm.at[cid, f, t, pl.ds(base_n, NT)],
                               rb[f % 2].at[pl.ds(t * NT, NT)], rsem)
              for t in range(NS)]

    rds = {0: fire_red(0)}
    for f in range(FC):
      if f + 1 < FC:
        rds[f + 1] = fire_red(f + 1)
      for d in rds.pop(f):
        d.wait()
      rbf = rb[f % 2]

      @plsc.parallel_loop(0, NT // 16, unroll=2)
      def red(v):
        ds16 = pl.ds(16 * v, 16)
        a = rbf[pl.ds(16 * v, 16)]
        for t in range(1, NS):
          a = a + rbf[pl.ds(t * NT + 16 * v, 16)]
        hn = ((1.0 - ALPHA) * (dis_v[ds16] * a
                               + dis2_v[ds16] * hown[f][ds16])
              + ALPHA * h0buf[f][ds16])
        hown[f][ds16] = hn
        hbuf[ds16] = dis_v[ds16] * hn
      pltpu.sync_copy(hbuf, u_s[f].at[own])

    plsc.subcore_barrier()

    for f in range(FC):
      pltpu.sync_copy(u_s[f], u_loc[f])
    return carry

  lax.fori_loop(0, ITERS, one_iter, 0)

  for f in range(FC):
    pltpu.sync_copy(hown[f], out_hbm.at[cid, f, own])


def _appnp(h0_t, rc_p, w_p):
  mesh = plsc.VectorSubcoreMesh(core_axis_name="c", subcore_axis_name="s",
                                num_cores=2, num_subcores=16)
  f = pl.kernel(
      _appnp_body,
      out_type=[jax.ShapeDtypeStruct((2, FC, NP), jnp.float32),
                jax.ShapeDtypeStruct((2, FC, NS, NP), jnp.float32)],
      mesh=mesh,
      compiler_params=pltpu.CompilerParams(needs_layout_passes=False),
      scratch_types=[
          pltpu.VMEM((CH,), jnp.int32),
          pltpu.VMEM((CH,), jnp.int32),
          pltpu.VMEM((CH,), jnp.float32),
          pltpu.VMEM((CH,), jnp.float32),
          pltpu.VMEM((CH,), jnp.int32),
          pltpu.VMEM((NP,), jnp.float32),
          pltpu.VMEM((NP,), jnp.float32),
          pltpu.VMEM((NP,), jnp.float32),
          pltpu.VMEM((NP,), jnp.float32),
          pltpu.VMEM((NP,), jnp.float32),
          pltpu.VMEM((NP,), jnp.float32),
          pltpu.VMEM((NP,), jnp.float32),
          pltpu.VMEM((NP,), jnp.float32),
          pltpu.VMEM((NS * NT,), jnp.float32),
          pltpu.VMEM((NS * NT,), jnp.float32),
          pltpu.VMEM((NT,), jnp.float32),
          pltpu.VMEM((NT,), jnp.float32),
          pltpu.VMEM((NT,), jnp.float32),
          pltpu.VMEM((NT,), jnp.float32),
          pltpu.VMEM((NT,), jnp.float32),
          pltpu.VMEM((NT,), jnp.float32),
          pltpu.VMEM((NT,), jnp.float32),
          pltpu.VMEM((NT,), jnp.float32),
          pltpu.VMEM((NT,), jnp.float32),
          pltpu.VMEM((NT,), jnp.float32),
          pltpu.VMEM((NT,), jnp.float32),
          pltpu.SemaphoreType.DMA,
          pltpu.SemaphoreType.DMA,
          pltpu.SemaphoreType.DMA,
          pltpu.SemaphoreType.DMA,
          pltpu.VMEM_SHARED((NP,), jnp.float32),
          pltpu.VMEM_SHARED((NP,), jnp.float32),
          pltpu.VMEM_SHARED((NP,), jnp.float32),
          pltpu.VMEM_SHARED((NP,), jnp.float32),
          pltpu.VMEM_SHARED((NP,), jnp.float32),
          pltpu.VMEM_SHARED((NP,), jnp.float32),
      ],
  )
  return f(h0_t, rc_p, w_p)


@jax.jit
def kernel(x, edge_index, edge_weight, W1, b1, W2, b2):
  x_p = jnp.pad(x, ((0, NP - N), (0, 0)))
  W2p = jnp.pad(W2, ((0, 0), (0, FP - W2.shape[1])))
  b2p = jnp.pad(b2, (0, FP - b2.shape[0]))

  h0 = _mlp(x_p, W1, b1, W2p, b2p)
  h0_t = h0.T.reshape(2, FC, NP)

  npad = EP - E
  pad_idx = (jnp.arange(npad, dtype=jnp.int32) * 131) % N
  row_p = jnp.concatenate([edge_index[0], pad_idx])
  col_p = jnp.concatenate([edge_index[1], pad_idx])
  rc_p = (row_p * 16384 + col_p).reshape(16, NCH, CH)
  w_p = jnp.concatenate(
      [edge_weight, jnp.zeros((npad,), jnp.float32)]).reshape(16, NCH, CH)

  out, _partials = _appnp(h0_t, rc_p, w_p)
  out = out.reshape(FP, NP).T
  return out[:N, :6]


# speedup vs baseline: 2.1186x; 1.0249x over previous
import jax
import jax.numpy as jnp
from jax import lax
from jax.experimental import pallas as pl
from jax.experimental.pallas import tpu as pltpu
from jax.experimental.pallas import tpu_sc as plsc

N = 10000
NP = 10240
NT = 640
FP = 8
FC = 4
E = 320000
EP = 327680
ET = 20480
CH = 2048
NCH = ET // CH
ALPHA = 0.1
ITERS = 10
DF = 128
DH = 64
NS = 16


def _mlp_body(x_ref, w1_ref, b1_ref, w2_ref, b2_ref, o_ref):
  h = jnp.dot(x_ref[...], w1_ref[...], preferred_element_type=jnp.float32)
  h = jnp.maximum(h + b1_ref[...], 0.0)
  o = jnp.dot(h, w2_ref[...], preferred_element_type=jnp.float32)
  o_ref[...] = o + b2_ref[...]


def _mlp(x_p, W1, b1, W2p, b2p):
  BM = 1024
  return pl.pallas_call(
      _mlp_body,
      grid=(NP // BM,),
      in_specs=[
          pl.BlockSpec((BM, DF), lambda i: (i, 0)),
          pl.BlockSpec((DF, DH), lambda i: (0, 0)),
          pl.BlockSpec((1, DH), lambda i: (0, 0)),
          pl.BlockSpec((DH, FP), lambda i: (0, 0)),
          pl.BlockSpec((1, FP), lambda i: (0, 0)),
      ],
      out_specs=pl.BlockSpec((BM, FP), lambda i: (i, 0)),
      out_shape=jax.ShapeDtypeStruct((NP, FP), jnp.float32),
  )(x_p, W1, b1.reshape(1, DH), W2p, b2p.reshape(1, FP))


def _rsqrt_newton(x):
  y = 1.0 / x
  for _ in range(28):
    y = y * (1.5 - 0.5 * x * y * y)
  return y


def _appnp_body(h0_hbm, rc_hbm, w_hbm, out_hbm, p_hbm,
                rcb0, rcb1, wb0, wb1, rowb,
                ul0, ul1, ul2, ul3, al0, al1, al2, al3,
                rbuf, rbuf2, ob0, ob1, ob2, ob3, hw0, hw1, hw2, hw3,
                hbuf, dis_v, dis2_v,
                ld0, ld1, psem, rsem,
                u_s0, u_s1, u_s2, u_s3, deg_s, dis_s):
  rcb = [rcb0, rcb1]
  wb = [wb0, wb1]
  u_loc = [ul0, ul1, ul2, ul3]
  a_loc = [al0, al1, al2, al3]
  h0buf = [ob0, ob1, ob2, ob3]
  hown = [hw0, hw1, hw2, hw3]
  u_s = [u_s0, u_s1, u_s2, u_s3]
  lsem = [ld0, ld1]
  cid = lax.axis_index("c")
  sid = lax.axis_index("s")
  base_n = sid * NT
  own = pl.ds(base_n, NT)

  zeros16 = jnp.zeros((16,), jnp.float32)

  def fire_load(c, b):
    return [pltpu.async_copy(rc_hbm.at[sid, c], rcb[b], lsem[b]),
            pltpu.async_copy(w_hbm.at[sid, c], wb[b], lsem[b])]

  @plsc.parallel_loop(0, NP // 16, unroll=4)
  def azero(v):
    for f in range(FC):
      a_loc[f][pl.ds(16 * v, 16)] = zeros16

  for f in range(FC):
    pltpu.sync_copy(h0_hbm.at[cid, f, own], h0buf[f])
  pltpu.sync_copy(al0.at[pl.ds(0, NT)], deg_s.at[own])
  plsc.subcore_barrier()

  pd = {0: fire_load(0, 0)}
  for c in range(NCH):
    if c + 1 < NCH:
      pd[c + 1] = fire_load(c + 1, (c + 1) % 2)
    for d in pd.pop(c):
      d.wait()
    b = c % 2

    @plsc.parallel_loop(0, CH // 16, unroll=4)
    def rowex(v):
      rowb[pl.ds(16 * v, 16)] = lax.shift_right_logical(
          rcb[b][pl.ds(16 * v, 16)], 14)
    pltpu.sync_copy(wb[b], deg_s.at[rowb], add=True)
  plsc.subcore_barrier()

  pltpu.sync_copy(deg_s.at[own], hbuf)

  def dis_calc(v, _):
    dv = hbuf[pl.ds(16 * v, 16)] + 1.0
    d = _rsqrt_newton(dv)
    dis_v[pl.ds(16 * v, 16)] = d
    dis2_v[pl.ds(16 * v, 16)] = d * d
    return 0
  lax.fori_loop(0, NT // 16, dis_calc, 0)

  pltpu.sync_copy(dis_v, dis_s.at[own])

  for f in range(FC):
    def u_init(v, _):
      ds16 = pl.ds(16 * v, 16)
      h0v = h0buf[f][ds16]
      hown[f][ds16] = h0v
      hbuf[ds16] = dis_v[ds16] * h0v
      return 0
    lax.fori_loop(0, NT // 16, u_init, 0)
    pltpu.sync_copy(hbuf, u_s[f].at[own])
  plsc.subcore_barrier()

  for f in range(FC):
    pltpu.sync_copy(u_s[f], u_loc[f])

  def one_iter(_, carry):
    pd = {0: fire_load(0, 0)}
    for c in range(NCH):
      if c + 1 < NCH:
        pd[c + 1] = fire_load(c + 1, (c + 1) % 2)
      for d in pd.pop(c):
        d.wait()
      b = c % 2

      @plsc.parallel_loop(0, CH // 64, unroll=4)
      def edge(v0):
        for u in range(4):
          ds16 = pl.ds(64 * v0 + 16 * u, 16)
          rc = rcb[b][ds16]
          wv = wb[b][ds16]
          row = lax.shift_right_logical(rc, 14)
          col = lax.bitwise_and(rc, 16383)
          for f in range(FC):
            uv = plsc.load_gather(u_loc[f], [col])
            plsc.addupdate_scatter(a_loc[f], [row], uv * wv)

    wd = [pltpu.async_copy(a_loc[f], p_hbm.at[cid, f, sid], psem)
          for f in range(FC)]
    for d in wd:
      d.wait()

    @plsc.parallel_loop(0, NP // 16, unroll=4)
    def azero2(v):
      for f in range(FC):
        a_loc[f][pl.ds(16 * v, 16)] = zeros16
    plsc.subcore_barrier()

    rb = [rbuf, rbuf2]

    def fire_red(f):
      return [pltpu.async_copy(p_hbm.at[cid, f, t, pl.ds(base_n, NT)],
                               rb[f % 2].at[pl.ds(t * NT, NT)], rsem)
              for t in range(NS)]

    rds = {0: fire_red(0)}
    for f in range(FC):
      if f + 1 < FC:
        rds[f + 1] = fire_red(f + 1)
      for d in rds.pop(f):
        d.wait()
      rbf = rb[f % 2]

      @plsc.parallel_loop(0, NT // 16, unroll=2)
      def red(v):
        ds16 = pl.ds(16 * v, 16)
        a = rbf[pl.ds(16 * v, 16)]
        for t in range(1, NS):
          a = a + rbf[pl.ds(t * NT + 16 * v, 16)]
        hn = ((1.0 - ALPHA) * (dis_v[ds16] * a
                               + dis2_v[ds16] * hown[f][ds16])
              + ALPHA * h0buf[f][ds16])
        hown[f][ds16] = hn
        hbuf[ds16] = dis_v[ds16] * hn
      pltpu.sync_copy(hbuf, u_s[f].at[own])

    plsc.subcore_barrier()

    fd = [pltpu.async_copy(u_s[f], u_loc[f], rsem) for f in range(FC)]
    for d in fd:
      d.wait()
    return carry

  lax.fori_loop(0, ITERS, one_iter, 0)

  for f in range(FC):
    pltpu.sync_copy(hown[f], out_hbm.at[cid, f, own])


def _appnp(h0_t, rc_p, w_p):
  mesh = plsc.VectorSubcoreMesh(core_axis_name="c", subcore_axis_name="s",
                                num_cores=2, num_subcores=16)
  f = pl.kernel(
      _appnp_body,
      out_type=[jax.ShapeDtypeStruct((2, FC, NP), jnp.float32),
                jax.ShapeDtypeStruct((2, FC, NS, NP), jnp.float32)],
      mesh=mesh,
      compiler_params=pltpu.CompilerParams(needs_layout_passes=False),
      scratch_types=[
          pltpu.VMEM((CH,), jnp.int32),
          pltpu.VMEM((CH,), jnp.int32),
          pltpu.VMEM((CH,), jnp.float32),
          pltpu.VMEM((CH,), jnp.float32),
          pltpu.VMEM((CH,), jnp.int32),
          pltpu.VMEM((NP,), jnp.float32),
          pltpu.VMEM((NP,), jnp.float32),
          pltpu.VMEM((NP,), jnp.float32),
          pltpu.VMEM((NP,), jnp.float32),
          pltpu.VMEM((NP,), jnp.float32),
          pltpu.VMEM((NP,), jnp.float32),
          pltpu.VMEM((NP,), jnp.float32),
          pltpu.VMEM((NP,), jnp.float32),
          pltpu.VMEM((NS * NT,), jnp.float32),
          pltpu.VMEM((NS * NT,), jnp.float32),
          pltpu.VMEM((NT,), jnp.float32),
          pltpu.VMEM((NT,), jnp.float32),
          pltpu.VMEM((NT,), jnp.float32),
          pltpu.VMEM((NT,), jnp.float32),
          pltpu.VMEM((NT,), jnp.float32),
          pltpu.VMEM((NT,), jnp.float32),
          pltpu.VMEM((NT,), jnp.float32),
          pltpu.VMEM((NT,), jnp.float32),
          pltpu.VMEM((NT,), jnp.float32),
          pltpu.VMEM((NT,), jnp.float32),
          pltpu.VMEM((NT,), jnp.float32),
          pltpu.SemaphoreType.DMA,
          pltpu.SemaphoreType.DMA,
          pltpu.SemaphoreType.DMA,
          pltpu.SemaphoreType.DMA,
          pltpu.VMEM_SHARED((NP,), jnp.float32),
          pltpu.VMEM_SHARED((NP,), jnp.float32),
          pltpu.VMEM_SHARED((NP,), jnp.float32),
          pltpu.VMEM_SHARED((NP,), jnp.float32),
          pltpu.VMEM_SHARED((NP,), jnp.float32),
          pltpu.VMEM_SHARED((NP,), jnp.float32),
      ],
  )
  return f(h0_t, rc_p, w_p)


@jax.jit
def kernel(x, edge_index, edge_weight, W1, b1, W2, b2):
  x_p = jnp.pad(x, ((0, NP - N), (0, 0)))
  W2p = jnp.pad(W2, ((0, 0), (0, FP - W2.shape[1])))
  b2p = jnp.pad(b2, (0, FP - b2.shape[0]))

  h0 = _mlp(x_p, W1, b1, W2p, b2p)
  h0_t = h0.T.reshape(2, FC, NP)

  npad = EP - E
  pad_idx = (jnp.arange(npad, dtype=jnp.int32) * 131) % N
  row_p = jnp.concatenate([edge_index[0], pad_idx])
  col_p = jnp.concatenate([edge_index[1], pad_idx])
  rc_p = (row_p * 16384 + col_p).reshape(16, NCH, CH)
  w_p = jnp.concatenate(
      [edge_weight, jnp.zeros((npad,), jnp.float32)]).reshape(16, NCH, CH)

  out, _partials = _appnp(h0_t, rc_p, w_p)
  out = out.reshape(FP, NP).T
  return out[:N, :6]


# speedup vs baseline: 2.1348x; 1.0076x over previous
import jax
import jax.numpy as jnp
from jax import lax
from jax.experimental import pallas as pl
from jax.experimental.pallas import tpu as pltpu
from jax.experimental.pallas import tpu_sc as plsc

N = 10000
NP = 10240
NT = 640
FP = 8
FC = 4
E = 320000
EP = 327680
ET = 20480
CH = 2048
NCH = ET // CH
ALPHA = 0.1
ITERS = 10
DF = 128
DH = 64
NS = 16


def _mlp_body(x_ref, w1_ref, b1_ref, w2_ref, b2_ref, o_ref):
  h = jnp.dot(x_ref[...], w1_ref[...], preferred_element_type=jnp.float32)
  h = jnp.maximum(h + b1_ref[...], 0.0)
  o = jnp.dot(h, w2_ref[...], preferred_element_type=jnp.float32)
  o_ref[...] = o + b2_ref[...]


def _mlp(x_p, W1, b1, W2p, b2p):
  BM = 1024
  return pl.pallas_call(
      _mlp_body,
      grid=(NP // BM,),
      in_specs=[
          pl.BlockSpec((BM, DF), lambda i: (i, 0)),
          pl.BlockSpec((DF, DH), lambda i: (0, 0)),
          pl.BlockSpec((1, DH), lambda i: (0, 0)),
          pl.BlockSpec((DH, FP), lambda i: (0, 0)),
          pl.BlockSpec((1, FP), lambda i: (0, 0)),
      ],
      out_specs=pl.BlockSpec((BM, FP), lambda i: (i, 0)),
      out_shape=jax.ShapeDtypeStruct((NP, FP), jnp.float32),
  )(x_p, W1, b1.reshape(1, DH), W2p, b2p.reshape(1, FP))


def _rsqrt_newton(x):
  y = 1.0 / x
  for _ in range(28):
    y = y * (1.5 - 0.5 * x * y * y)
  return y


def _appnp_body(h0_hbm, rc_hbm, w_hbm, out_hbm, p_hbm,
                rcb0, rcb1, wb0, wb1, rowb,
                ul0, ul1, ul2, ul3, al0, al1, al2, al3,
                rbuf, rbuf2, ob0, ob1, ob2, ob3, hw0, hw1, hw2, hw3,
                hbuf, ub0, ub1, ub2, ub3, dis_v, dis2_v,
                ld0, ld1, psem, rsem,
                u_s0, u_s1, u_s2, u_s3, deg_s, dis_s):
  rcb = [rcb0, rcb1]
  wb = [wb0, wb1]
  u_loc = [ul0, ul1, ul2, ul3]
  a_loc = [al0, al1, al2, al3]
  h0buf = [ob0, ob1, ob2, ob3]
  hown = [hw0, hw1, hw2, hw3]
  ubuf = [ub0, ub1, ub2, ub3]
  u_s = [u_s0, u_s1, u_s2, u_s3]
  lsem = [ld0, ld1]
  cid = lax.axis_index("c")
  sid = lax.axis_index("s")
  base_n = sid * NT
  own = pl.ds(base_n, NT)

  zeros16 = jnp.zeros((16,), jnp.float32)

  def fire_load(c, b):
    return [pltpu.async_copy(rc_hbm.at[sid, c], rcb[b], lsem[b]),
            pltpu.async_copy(w_hbm.at[sid, c], wb[b], lsem[b])]

  @plsc.parallel_loop(0, NP // 16, unroll=4)
  def azero(v):
    for f in range(FC):
      a_loc[f][pl.ds(16 * v, 16)] = zeros16

  for f in range(FC):
    pltpu.sync_copy(h0_hbm.at[cid, f, own], h0buf[f])
  pltpu.sync_copy(al0.at[pl.ds(0, NT)], deg_s.at[own])
  plsc.subcore_barrier()

  pd = {0: fire_load(0, 0)}
  for c in range(NCH):
    if c + 1 < NCH:
      pd[c + 1] = fire_load(c + 1, (c + 1) % 2)
    for d in pd.pop(c):
      d.wait()
    b = c % 2

    @plsc.parallel_loop(0, CH // 16, unroll=4)
    def rowex(v):
      rowb[pl.ds(16 * v, 16)] = lax.shift_right_logical(
          rcb[b][pl.ds(16 * v, 16)], 14)
    pltpu.sync_copy(wb[b], deg_s.at[rowb], add=True)
  plsc.subcore_barrier()

  pltpu.sync_copy(deg_s.at[own], hbuf)

  def dis_calc(v, _):
    dv = hbuf[pl.ds(16 * v, 16)] + 1.0
    d = _rsqrt_newton(dv)
    dis_v[pl.ds(16 * v, 16)] = d
    dis2_v[pl.ds(16 * v, 16)] = d * d
    return 0
  lax.fori_loop(0, NT // 16, dis_calc, 0)

  pltpu.sync_copy(dis_v, dis_s.at[own])

  for f in range(FC):
    def u_init(v, _):
      ds16 = pl.ds(16 * v, 16)
      h0v = h0buf[f][ds16]
      hown[f][ds16] = h0v
      hbuf[ds16] = dis_v[ds16] * h0v
      return 0
    lax.fori_loop(0, NT // 16, u_init, 0)
    pltpu.sync_copy(hbuf, u_s[f].at[own])
  plsc.subcore_barrier()

  for f in range(FC):
    pltpu.sync_copy(u_s[f], u_loc[f])

  def one_iter(_, carry):
    pd = {0: fire_load(0, 0)}
    for c in range(NCH):
      if c + 1 < NCH:
        pd[c + 1] = fire_load(c + 1, (c + 1) % 2)
      for d in pd.pop(c):
        d.wait()
      b = c % 2

      @plsc.parallel_loop(0, CH // 64, unroll=4)
      def edge(v0):
        for u in range(4):
          ds16 = pl.ds(64 * v0 + 16 * u, 16)
          rc = rcb[b][ds16]
          wv = wb[b][ds16]
          row = lax.shift_right_logical(rc, 14)
          col = lax.bitwise_and(rc, 16383)
          for f in range(FC):
            uv = plsc.load_gather(u_loc[f], [col])
            plsc.addupdate_scatter(a_loc[f], [row], uv * wv)

    wd = [pltpu.async_copy(a_loc[f], p_hbm.at[cid, f, sid], psem)
          for f in range(FC)]
    for d in wd:
      d.wait()

    @plsc.parallel_loop(0, NP // 16, unroll=4)
    def azero2(v):
      for f in range(FC):
        a_loc[f][pl.ds(16 * v, 16)] = zeros16
    plsc.subcore_barrier()

    rb = [rbuf, rbuf2]

    def fire_red(f):
      return [pltpu.async_copy(p_hbm.at[cid, f, t, pl.ds(base_n, NT)],
                               rb[f % 2].at[pl.ds(t * NT, NT)], rsem)
              for t in range(NS)]

    rds = {0: fire_red(0)}
    uw = []
    for f in range(FC):
      if f + 1 < FC:
        rds[f + 1] = fire_red(f + 1)
      for d in rds.pop(f):
        d.wait()
      rbf = rb[f % 2]

      @plsc.parallel_loop(0, NT // 16, unroll=2)
      def red(v):
        ds16 = pl.ds(16 * v, 16)
        a = rbf[pl.ds(16 * v, 16)]
        for t in range(1, NS):
          a = a + rbf[pl.ds(t * NT + 16 * v, 16)]
        hn = ((1.0 - ALPHA) * (dis_v[ds16] * a
                               + dis2_v[ds16] * hown[f][ds16])
              + ALPHA * h0buf[f][ds16])
        hown[f][ds16] = hn
        ubuf[f][ds16] = dis_v[ds16] * hn
      uw.append(pltpu.async_copy(ubuf[f], u_s[f].at[own], psem))

    for d in uw:
      d.wait()
    plsc.subcore_barrier()

    fd = [pltpu.async_copy(u_s[f], u_loc[f], rsem) for f in range(FC)]
    for d in fd:
      d.wait()
    return carry

  lax.fori_loop(0, ITERS, one_iter, 0)

  for f in range(FC):
    pltpu.sync_copy(hown[f], out_hbm.at[cid, f, own])


def _appnp(h0_t, rc_p, w_p):
  mesh = plsc.VectorSubcoreMesh(core_axis_name="c", subcore_axis_name="s",
                                num_cores=2, num_subcores=16)
  f = pl.kernel(
      _appnp_body,
      out_type=[jax.ShapeDtypeStruct((2, FC, NP), jnp.float32),
                jax.ShapeDtypeStruct((2, FC, NS, NP), jnp.float32)],
      mesh=mesh,
      compiler_params=pltpu.CompilerParams(needs_layout_passes=False),
      scratch_types=[
          pltpu.VMEM((CH,), jnp.int32),
          pltpu.VMEM((CH,), jnp.int32),
          pltpu.VMEM((CH,), jnp.float32),
          pltpu.VMEM((CH,), jnp.float32),
          pltpu.VMEM((CH,), jnp.int32),
          pltpu.VMEM((NP,), jnp.float32),
          pltpu.VMEM((NP,), jnp.float32),
          pltpu.VMEM((NP,), jnp.float32),
          pltpu.VMEM((NP,), jnp.float32),
          pltpu.VMEM((NP,), jnp.float32),
          pltpu.VMEM((NP,), jnp.float32),
          pltpu.VMEM((NP,), jnp.float32),
          pltpu.VMEM((NP,), jnp.float32),
          pltpu.VMEM((NS * NT,), jnp.float32),
          pltpu.VMEM((NS * NT,), jnp.float32),
          pltpu.VMEM((NT,), jnp.float32),
          pltpu.VMEM((NT,), jnp.float32),
          pltpu.VMEM((NT,), jnp.float32),
          pltpu.VMEM((NT,), jnp.float32),
          pltpu.VMEM((NT,), jnp.float32),
          pltpu.VMEM((NT,), jnp.float32),
          pltpu.VMEM((NT,), jnp.float32),
          pltpu.VMEM((NT,), jnp.float32),
          pltpu.VMEM((NT,), jnp.float32),
          pltpu.VMEM((NT,), jnp.float32),
          pltpu.VMEM((NT,), jnp.float32),
          pltpu.VMEM((NT,), jnp.float32),
          pltpu.VMEM((NT,), jnp.float32),
          pltpu.VMEM((NT,), jnp.float32),
          pltpu.VMEM((NT,), jnp.float32),
          pltpu.SemaphoreType.DMA,
          pltpu.SemaphoreType.DMA,
          pltpu.SemaphoreType.DMA,
          pltpu.SemaphoreType.DMA,
          pltpu.VMEM_SHARED((NP,), jnp.float32),
          pltpu.VMEM_SHARED((NP,), jnp.float32),
          pltpu.VMEM_SHARED((NP,), jnp.float32),
          pltpu.VMEM_SHARED((NP,), jnp.float32),
          pltpu.VMEM_SHARED((NP,), jnp.float32),
          pltpu.VMEM_SHARED((NP,), jnp.float32),
      ],
  )
  return f(h0_t, rc_p, w_p)


@jax.jit
def kernel(x, edge_index, edge_weight, W1, b1, W2, b2):
  x_p = jnp.pad(x, ((0, NP - N), (0, 0)))
  W2p = jnp.pad(W2, ((0, 0), (0, FP - W2.shape[1])))
  b2p = jnp.pad(b2, (0, FP - b2.shape[0]))

  h0 = _mlp(x_p, W1, b1, W2p, b2p)
  h0_t = h0.T.reshape(2, FC, NP)

  npad = EP - E
  pad_idx = (jnp.arange(npad, dtype=jnp.int32) * 131) % N
  row_p = jnp.concatenate([edge_index[0], pad_idx])
  col_p = jnp.concatenate([edge_index[1], pad_idx])
  rc_p = (row_p * 16384 + col_p).reshape(16, NCH, CH)
  w_p = jnp.concatenate(
      [edge_weight, jnp.zeros((npad,), jnp.float32)]).reshape(16, NCH, CH)

  out, _partials = _appnp(h0_t, rc_p, w_p)
  out = out.reshape(FP, NP).T
  return out[:N, :6]
